# baseline (speedup 1.0000x reference)
import jax
import jax.numpy as jnp
from jax.experimental import pallas as pl
from jax.experimental.pallas import tpu as pltpu

_B = 16384
_D = 128
_R = _B // 128
_BS = 64
_GRID = _R // _BS


def _fused_kernel(u_ref, d1_ref, d2_ref, v1_ref, v2_ref,
                  out_ref, w1s, w2s):
    i = pl.program_id(0)

    @pl.when(i == 0)
    def _():
        u = u_ref[...]
        d2 = d2_ref[...]
        m1 = jnp.max(u)
        m2 = jnp.max(u - d2)
        w2s[...] = jnp.minimum(d2, m1)
        w1s[...] = jnp.minimum(d1_ref[...], m2)

    w2 = w2s[pl.ds(i * _BS, _BS), :]
    w1 = w1s[pl.ds(i * _BS, _BS), :]
    out_ref[...] = (v2_ref[...] * w2[:, :, None]
                    + v1_ref[...] * w1[:, :, None])


def kernel(u, d1, d2, v1, v2):
    u_r = u.reshape(128, 128)
    d1_r = d1.reshape(128, 128)
    d2_r = d2.reshape(128, 128)
    v1_r = v1.reshape(128, 128, _D)
    v2_r = v2.reshape(128, 128, _D)
    out = pl.pallas_call(
        _fused_kernel,
        grid=(_GRID,),
        in_specs=[
            pl.BlockSpec((128, 128), lambda i: (0, 0)),
            pl.BlockSpec((128, 128), lambda i: (0, 0)),
            pl.BlockSpec((128, 128), lambda i: (0, 0)),
            pl.BlockSpec((_BS, 128, _D), lambda i: (i, 0, 0)),
            pl.BlockSpec((_BS, 128, _D), lambda i: (i, 0, 0)),
        ],
        out_specs=pl.BlockSpec((_BS, 128, _D), lambda i: (i, 0, 0)),
        out_shape=jax.ShapeDtypeStruct((128, 128, _D), jnp.float32),
        scratch_shapes=[
            pltpu.VMEM((128, 128), jnp.float32),
            pltpu.VMEM((128, 128), jnp.float32),
        ],
    )(u_r, d1_r, d2_r, v1_r, v2_r)
    return out.reshape(_B, _D)
